# baseline (speedup 1.0000x reference)
import functools

import jax
import jax.numpy as jnp
from jax import lax
from jax.experimental import pallas as pl
from jax.experimental.pallas import tpu as pltpu


def _sigmoid(x):
    return 1.0 / (1.0 + jnp.exp(-x))


def _sel_dot(x, E):
    xi = lax.bitcast_convert_type(x, jnp.uint32)
    hi = lax.bitcast_convert_type(xi & jnp.uint32(0xFFFF0000), jnp.float32)
    lo = x - hi
    return (jnp.dot(hi, E, preferred_element_type=jnp.float32)
            + jnp.dot(lo, E, preferred_element_type=jnp.float32))


def _softplus(x):
    return jnp.maximum(x, 0.0) + jnp.log(1.0 + jnp.exp(-jnp.abs(x)))


def _inproj_kernel(x_ref, w_ref, o_ref):
    o_ref[...] = jnp.dot(x_ref[...], w_ref[...],
                         preferred_element_type=jnp.float32)


def _inproj(x_bf16, w_bf16, *, tm=2048, tn=1024):
    M, K = x_bf16.shape
    _, N = w_bf16.shape
    grid_m = (M + tm - 1) // tm
    grid_n = (N + tn - 1) // tn
    return pl.pallas_call(
        _inproj_kernel,
        out_shape=jax.ShapeDtypeStruct((M, N), jnp.float32),
        grid=(grid_m, grid_n),
        in_specs=[
            pl.BlockSpec((tm, K), lambda i, j: (i, 0)),
            pl.BlockSpec((K, tn), lambda i, j: (0, j)),
        ],
        out_specs=pl.BlockSpec((tm, tn), lambda i, j: (i, j)),
        compiler_params=pltpu.CompilerParams(
            dimension_semantics=("parallel", "arbitrary"),
            vmem_limit_bytes=56 * 1024 * 1024),
    )(x_bf16, w_bf16)


def _conv_kernel(x_ref, w_ref, b_ref, o_ref):
    L, C = o_ref.shape[1], o_ref.shape[2]
    K = w_ref.shape[0]
    x = x_ref[0]
    w = w_ref[:, 0, :]
    acc = x * w[K - 1:K, :] + b_ref[...]
    for d in range(1, K):
        sh = jnp.concatenate(
            [jnp.zeros((d, C), jnp.float32), x[: L - d, :]], axis=0)
        acc = acc + sh * w[K - 1 - d:K - d, :]
    o_ref[0] = (acc * _sigmoid(acc)).astype(o_ref.dtype)


def _conv_silu(zxbcdt_3d, w_klc, conv_b, *, col0, conv_dim, cc=256):
    Bsz, L, _ = zxbcdt_3d.shape
    K = w_klc.shape[0]
    assert col0 % cc == 0 and conv_dim % cc == 0
    t0 = col0 // cc
    return pl.pallas_call(
        _conv_kernel,
        out_shape=jax.ShapeDtypeStruct((Bsz, L, conv_dim), jnp.bfloat16),
        grid=(Bsz, conv_dim // cc),
        in_specs=[
            pl.BlockSpec((1, L, cc), lambda b, c: (b, 0, t0 + c)),
            pl.BlockSpec((K, 1, cc), lambda b, c: (0, 0, c)),
            pl.BlockSpec((1, cc), lambda b, c: (0, c)),
        ],
        out_specs=pl.BlockSpec((1, L, cc), lambda b, c: (b, 0, c)),
        compiler_params=pltpu.CompilerParams(
            dimension_semantics=("parallel", "arbitrary")),
    )(zxbcdt_3d, w_klc, conv_b.reshape(1, conv_dim))


def _ssd_kernel(A_ref, dvp_ref, dtb_ref, dt_ref, xbc_ref, Bm_ref, Cm_ref,
                e1_ref, e2_ref, y_ref, state_ref, xw_ref, bd_ref, yacc_ref,
                *, headdim, hgroup):
    P = headdim
    HT = A_ref.shape[-1]
    Q = xbc_ref.shape[1]
    G = hgroup

    @pl.when(pl.program_id(2) == 0)
    def _init():
        state_ref[...] = jnp.zeros_like(state_ref)

    @pl.when((pl.program_id(1) == 0) & (pl.program_id(2) == 0))
    def _init_bd():
        bd_ref[...] = jnp.zeros_like(bd_ref)

    A = A_ref[0]
    DvP = dvp_ref[0]
    dtb = dtb_ref[0]
    dt_raw = dt_ref[0, 0]
    x = xbc_ref[0]
    Bg = Bm_ref[0]
    Cg = Cm_ref[0]
    E1 = e1_ref[...]
    E2 = e2_ref[...]

    dt = _softplus(dt_raw + dtb)
    a = dt * A

    idx_i = lax.broadcasted_iota(jnp.int32, (Q, Q), 0)
    idx_j = lax.broadcasted_iota(jnp.int32, (Q, Q), 1)
    causal = idx_i >= idx_j
    tri = causal.astype(jnp.float32)

    cA = jnp.dot(tri, a, preferred_element_type=jnp.float32)
    cAT = cA.T
    exp_cA = jnp.exp(cA)
    cA_last = cA[Q - 1:Q, :]
    w_all = jnp.exp(cA_last - cA) * dt

    dtP = _sel_dot(dt, E1)
    expP = _sel_dot(exp_cA, E1)
    wP = _sel_dot(w_all, E1)
    elP = expP[Q - 1:Q]
    M2 = _sel_dot(cA, E2)

    xf = x.astype(jnp.float32)
    xdt_bf = (xf * dtP).astype(jnp.bfloat16)
    xw_ref[...] = (xf * wP).astype(jnp.bfloat16)

    BgT = Bg.T
    scores = jnp.dot(Cg, BgT, preferred_element_type=jnp.float32)
    y_inter = jnp.dot(Cg, state_ref[...].astype(jnp.bfloat16),
                      preferred_element_type=jnp.float32)

    v = jnp.concatenate([cAT[h:h + 1, :] for h in range(HT)], axis=1)
    DIFF = M2 - v

    causalG = jnp.concatenate([causal] * G, axis=1)
    scoresG = pltpu.repeat(scores, G, axis=1)

    neg_big = jnp.float32(-1e30)
    for g in range(HT // G):
        dec = jnp.exp(jnp.where(causalG,
                                DIFF[:, g * G * Q:(g + 1) * G * Q], neg_big))
        lhs = (scoresG * dec).astype(jnp.bfloat16)
        for i in range(G):
            h = g * G + i
            bd_ref[i * Q:(i + 1) * Q, i * P:(i + 1) * P] = (
                xdt_bf[:, h * P:(h + 1) * P])
        yacc_ref[:, g * G * P:(g + 1) * G * P] = jnp.dot(
            lhs, bd_ref[...], preferred_element_type=jnp.float32)

    y_ref[0] = (yacc_ref[...] + expP * y_inter
                + DvP * xf).astype(y_ref.dtype)

    dS = jnp.dot(BgT, xw_ref[...], preferred_element_type=jnp.float32)
    state_ref[...] = elP * state_ref[...] + dS


def _ssd_scan(xbc, dt_t, A, Dv, dtb, *, nheads, headdim, d_state, chunk):
    Bsz, L, _ = xbc.shape
    H, P, N, Q = nheads, headdim, d_state, chunk
    d_inner = H * P
    HT = dt_t.shape[-1]
    n_tiles = H // HT
    nC = L // Q
    bcol = d_inner // (HT * P)
    assert d_inner % (HT * P) == 0 and L % Q == 0

    hh = jnp.arange(HT, dtype=jnp.int32)[:, None]
    E1 = (jnp.arange(HT * P, dtype=jnp.int32)[None, :] // P
          == hh).astype(jnp.float32)
    E2 = (jnp.arange(HT * Q, dtype=jnp.int32)[None, :] // Q
          == hh).astype(jnp.float32)
    DvP = jnp.repeat(Dv.reshape(n_tiles, 1, HT), P, axis=2)

    G = 4 if HT % 4 == 0 else (2 if HT % 2 == 0 else 1)
    kfn = functools.partial(_ssd_kernel, headdim=P, hgroup=G)
    return pl.pallas_call(
        kfn,
        out_shape=jax.ShapeDtypeStruct((Bsz, L, d_inner), jnp.bfloat16),
        grid=(Bsz, n_tiles, nC),
        in_specs=[
            pl.BlockSpec((1, 1, HT), lambda b, t, c: (t, 0, 0)),
            pl.BlockSpec((1, 1, HT * P), lambda b, t, c: (t, 0, 0)),
            pl.BlockSpec((1, 1, HT), lambda b, t, c: (t, 0, 0)),
            pl.BlockSpec((1, 1, Q, HT), lambda b, t, c: (b, t, c, 0)),
            pl.BlockSpec((1, Q, HT * P), lambda b, t, c: (b, c, t)),
            pl.BlockSpec((1, Q, N),
                         lambda b, t, c: (b, c, bcol * (HT * P) // N)),
            pl.BlockSpec((1, Q, N),
                         lambda b, t, c: (b, c, bcol * (HT * P) // N + 1)),
            pl.BlockSpec((HT, HT * P), lambda b, t, c: (0, 0)),
            pl.BlockSpec((HT, HT * Q), lambda b, t, c: (0, 0)),
        ],
        out_specs=pl.BlockSpec((1, Q, HT * P), lambda b, t, c: (b, c, t)),
        scratch_shapes=[pltpu.VMEM((N, HT * P), jnp.float32),
                        pltpu.VMEM((Q, HT * P), jnp.bfloat16),
                        pltpu.VMEM((G * Q, G * P), jnp.bfloat16),
                        pltpu.VMEM((Q, HT * P), jnp.float32)],
        compiler_params=pltpu.CompilerParams(
            dimension_semantics=("parallel", "arbitrary", "arbitrary"),
            vmem_limit_bytes=24 * 1024 * 1024),
    )(A, DvP, dtb, dt_t, xbc, xbc, xbc, E1, E2)


def _norm_proj_kernel(y_ref, z_ref, nw_ref, w_ref, o_ref):
    y = y_ref[...]
    z = z_ref[...]
    x = y * (z * _sigmoid(z))
    var = jnp.mean(x * x, axis=-1, keepdims=True)
    xn = x * lax.rsqrt(var + 1e-5) * nw_ref[...]
    o_ref[...] = jnp.dot(xn.astype(jnp.bfloat16), w_ref[...],
                         preferred_element_type=jnp.float32)


def _norm_proj(y2d, z_src, norm_w, w_bf16, *, tm=256):
    M, D = y2d.shape
    _, N = w_bf16.shape
    return pl.pallas_call(
        _norm_proj_kernel,
        out_shape=jax.ShapeDtypeStruct((M, N), jnp.float32),
        grid=(M // tm,),
        in_specs=[
            pl.BlockSpec((tm, D), lambda i: (i, 0)),
            pl.BlockSpec((tm, D), lambda i: (i, 0)),
            pl.BlockSpec((1, D), lambda i: (0, 0)),
            pl.BlockSpec((D, N), lambda i: (0, 0)),
        ],
        out_specs=pl.BlockSpec((tm, N), lambda i: (i, 0)),
        compiler_params=pltpu.CompilerParams(
            dimension_semantics=("parallel",),
            vmem_limit_bytes=56 * 1024 * 1024),
    )(y2d, z_src, norm_w.reshape(1, D), w_bf16)


def kernel(u, in_proj_wT, conv_w_klc, conv_b, A_log, D, dt_bias, norm_w,
           out_proj_wT):
    d_model, d_inner, d_state = 2048, 4096, 128
    H, P, G, K = 64, 64, 1, 4
    HT = 16
    chunk = 128
    n_tiles = H // HT
    conv_dim = d_inner + 2 * G * d_state
    d_in_proj = 2 * d_inner + 2 * G * d_state + H

    Bsz, L, _ = u.shape
    M = Bsz * L

    zxbcdt = _inproj(u.reshape(M, d_model).astype(jnp.bfloat16), in_proj_wT)

    xbc = _conv_silu(zxbcdt.reshape(Bsz, L, d_in_proj), conv_w_klc, conv_b,
                     col0=d_inner, conv_dim=conv_dim)

    dt_raw = zxbcdt[:, d_inner + conv_dim:]
    dt_t = dt_raw.reshape(Bsz, L, n_tiles, HT).transpose(0, 2, 1, 3)

    A = (-jnp.exp(A_log)).reshape(n_tiles, 1, HT).astype(jnp.float32)
    Dv = D.reshape(n_tiles, 1, HT).astype(jnp.float32)
    dtb = dt_bias.reshape(n_tiles, 1, HT).astype(jnp.float32)

    y = _ssd_scan(xbc, dt_t, A, Dv, dtb, nheads=H, headdim=P,
                  d_state=d_state, chunk=chunk)

    out = _norm_proj(y.reshape(M, d_inner), zxbcdt, norm_w, out_proj_wT)
    return out.reshape(Bsz, L, d_model)


# speedup vs baseline: 1.1053x; 1.1053x over previous
import functools

import jax
import jax.numpy as jnp
from jax import lax
from jax.experimental import pallas as pl
from jax.experimental.pallas import tpu as pltpu


def _sigmoid(x):
    return 1.0 / (1.0 + jnp.exp(-x))


def _sel_dot(x, E):
    xi = lax.bitcast_convert_type(x, jnp.uint32)
    hi = lax.bitcast_convert_type(xi & jnp.uint32(0xFFFF0000), jnp.float32)
    lo = x - hi
    return (jnp.dot(hi, E, preferred_element_type=jnp.float32)
            + jnp.dot(lo, E, preferred_element_type=jnp.float32))


def _softplus(x):
    return jnp.maximum(x, 0.0) + jnp.log(1.0 + jnp.exp(-jnp.abs(x)))


def _inproj_conv_kernel(x_ref, w_ref, cw_ref, cb_ref,
                        oz_ref, oxbc_ref, odt_ref, *,
                        nz_tiles, n_tiles, seq_len, dt_off):
    j = pl.program_id(1)
    acc = jnp.dot(x_ref[...], w_ref[...],
                  preferred_element_type=jnp.float32)

    @pl.when(j < nz_tiles)
    def _store_z():
        oz_ref[...] = acc

    @pl.when(j >= nz_tiles)
    def _conv_silu():
        R, C = acc.shape
        K = cw_ref.shape[0]
        cw = cw_ref[:, 0, :]
        rowmod = lax.broadcasted_iota(jnp.int32, (R, C), 0) & (seq_len - 1)
        total = acc * cw[K - 1:K, :] + cb_ref[...]
        for d in range(1, K):
            sh = jnp.concatenate(
                [jnp.zeros((d, C), jnp.float32), acc[: R - d, :]], axis=0)
            sh = jnp.where(rowmod >= d, sh, 0.0)
            total = total + sh * cw[K - 1 - d:K - d, :]
        oxbc_ref[...] = (total * _sigmoid(total)).astype(oxbc_ref.dtype)

    @pl.when(j == n_tiles - 1)
    def _store_dt():
        odt_ref[...] = acc[:, dt_off:dt_off + odt_ref.shape[1]]


def _inproj_conv(x_bf16, w_bf16, cwp, cbp, *, d_inner, seq_len,
                 tm=2048, tn=512):
    M, K = x_bf16.shape
    _, N = w_bf16.shape
    assert tm % seq_len == 0 and d_inner % tn == 0
    grid_m = M // tm
    grid_n = (N + tn - 1) // tn
    nz = d_inner // tn
    nx = grid_n - nz
    dt_off = (N - 64) - (grid_n - 1) * tn
    kfn = functools.partial(_inproj_conv_kernel, nz_tiles=nz,
                            n_tiles=grid_n, seq_len=seq_len, dt_off=dt_off)
    return pl.pallas_call(
        kfn,
        out_shape=[
            jax.ShapeDtypeStruct((M, d_inner), jnp.float32),
            jax.ShapeDtypeStruct((M, nx * tn), jnp.bfloat16),
            jax.ShapeDtypeStruct((M, 128), jnp.float32),
        ],
        grid=(grid_m, grid_n),
        in_specs=[
            pl.BlockSpec((tm, K), lambda i, j: (i, 0)),
            pl.BlockSpec((K, tn), lambda i, j: (0, j)),
            pl.BlockSpec((4, 1, tn),
                         lambda i, j, nz=nz: (0, 0, jnp.maximum(j - nz, 0))),
            pl.BlockSpec((1, tn),
                         lambda i, j, nz=nz: (0, jnp.maximum(j - nz, 0))),
        ],
        out_specs=[
            pl.BlockSpec((tm, tn),
                         lambda i, j, nz=nz: (i, jnp.minimum(j, nz - 1))),
            pl.BlockSpec((tm, tn),
                         lambda i, j, nz=nz: (i, jnp.maximum(j - nz, 0))),
            pl.BlockSpec((tm, 128), lambda i, j: (i, 0)),
        ],
        compiler_params=pltpu.CompilerParams(
            dimension_semantics=("parallel", "arbitrary"),
            vmem_limit_bytes=56 * 1024 * 1024),
    )(x_bf16, w_bf16, cwp, cbp)


def _ssd_kernel(A_ref, dvp_ref, dtb_ref, dt_ref, xbc_ref, Bm_ref, Cm_ref,
                e1_ref, e2_ref, y_ref, state_ref, xw_ref, *, headdim):
    P = headdim
    HT = A_ref.shape[-1]
    Q = xbc_ref.shape[1]

    @pl.when(pl.program_id(2) == 0)
    def _init():
        state_ref[...] = jnp.zeros_like(state_ref)

    A = A_ref[0]
    DvP = dvp_ref[0]
    dtb = dtb_ref[0]
    dt_raw = dt_ref[0, 0]
    x = xbc_ref[0]
    Bg = Bm_ref[0]
    Cg = Cm_ref[0]
    E1 = e1_ref[...]
    E2 = e2_ref[...]

    dt = _softplus(dt_raw + dtb)
    a = dt * A

    idx_i = lax.broadcasted_iota(jnp.int32, (Q, Q), 0)
    idx_j = lax.broadcasted_iota(jnp.int32, (Q, Q), 1)
    causal = idx_i >= idx_j
    tri = causal.astype(jnp.float32)

    cA = jnp.dot(tri, a, preferred_element_type=jnp.float32)
    cAT = cA.T
    exp_cA = jnp.exp(cA)
    cA_last = cA[Q - 1:Q, :]
    w_all = jnp.exp(cA_last - cA) * dt

    dtP = _sel_dot(dt, E1)
    expP = _sel_dot(exp_cA, E1)
    wP = _sel_dot(w_all, E1)
    elP = expP[Q - 1:Q]
    M2 = _sel_dot(cA, E2)

    xf = x.astype(jnp.float32)
    xdt_bf = (xf * dtP).astype(jnp.bfloat16)
    xw_ref[...] = (xf * wP).astype(jnp.bfloat16)

    BgT = Bg.T
    scores = jnp.dot(Cg, BgT, preferred_element_type=jnp.float32)
    y_inter = jnp.dot(Cg, state_ref[...].astype(jnp.bfloat16),
                      preferred_element_type=jnp.float32)

    neg_big = jnp.float32(-1e30)
    for h in range(HT):
        sl = slice(h * P, (h + 1) * P)
        sq = slice(h * Q, (h + 1) * Q)
        diff = M2[:, sq] - cAT[h:h + 1, :]
        dec = jnp.exp(jnp.where(causal, diff, neg_big))
        y_ref[0, :, sl] = jnp.dot((scores * dec).astype(jnp.bfloat16),
                                  xdt_bf[:, sl],
                                  preferred_element_type=jnp.float32)

    y_ref[0] = y_ref[0] + expP * y_inter + DvP * xf

    dS = jnp.dot(BgT, xw_ref[...], preferred_element_type=jnp.float32)
    state_ref[...] = elP * state_ref[...] + dS


def _ssd_scan(xbc, dt_t, A, Dv, dtb, *, nheads, headdim, d_state, chunk):
    Bsz, L, _ = xbc.shape
    H, P, N, Q = nheads, headdim, d_state, chunk
    d_inner = H * P
    HT = dt_t.shape[-1]
    n_tiles = H // HT
    nC = L // Q
    bcol = d_inner // (HT * P)
    assert d_inner % (HT * P) == 0 and L % Q == 0

    hh = jnp.arange(HT, dtype=jnp.int32)[:, None]
    E1 = (jnp.arange(HT * P, dtype=jnp.int32)[None, :] // P
          == hh).astype(jnp.float32)
    E2 = (jnp.arange(HT * Q, dtype=jnp.int32)[None, :] // Q
          == hh).astype(jnp.float32)
    DvP = jnp.repeat(Dv.reshape(n_tiles, 1, HT), P, axis=2)

    kfn = functools.partial(_ssd_kernel, headdim=P)
    return pl.pallas_call(
        kfn,
        out_shape=jax.ShapeDtypeStruct((Bsz, L, d_inner), jnp.float32),
        grid=(Bsz, n_tiles, nC),
        in_specs=[
            pl.BlockSpec((1, 1, HT), lambda b, t, c: (t, 0, 0)),
            pl.BlockSpec((1, 1, HT * P), lambda b, t, c: (t, 0, 0)),
            pl.BlockSpec((1, 1, HT), lambda b, t, c: (t, 0, 0)),
            pl.BlockSpec((1, 1, Q, HT), lambda b, t, c: (b, t, c, 0)),
            pl.BlockSpec((1, Q, HT * P), lambda b, t, c: (b, c, t)),
            pl.BlockSpec((1, Q, N),
                         lambda b, t, c: (b, c, bcol * (HT * P) // N)),
            pl.BlockSpec((1, Q, N),
                         lambda b, t, c: (b, c, bcol * (HT * P) // N + 1)),
            pl.BlockSpec((HT, HT * P), lambda b, t, c: (0, 0)),
            pl.BlockSpec((HT, HT * Q), lambda b, t, c: (0, 0)),
        ],
        out_specs=pl.BlockSpec((1, Q, HT * P), lambda b, t, c: (b, c, t)),
        scratch_shapes=[pltpu.VMEM((N, HT * P), jnp.float32),
                        pltpu.VMEM((Q, HT * P), jnp.bfloat16)],
        compiler_params=pltpu.CompilerParams(
            dimension_semantics=("parallel", "arbitrary", "arbitrary"),
            vmem_limit_bytes=24 * 1024 * 1024),
    )(A, DvP, dtb, dt_t, xbc, xbc, xbc, E1, E2)


def _norm_proj_kernel(y_ref, z_ref, nw_ref, w_ref, o_ref):
    y = y_ref[...]
    z = z_ref[...]
    x = y * (z * _sigmoid(z))
    var = jnp.mean(x * x, axis=-1, keepdims=True)
    xn = x * lax.rsqrt(var + 1e-5) * nw_ref[...]
    o_ref[...] = jnp.dot(xn.astype(jnp.bfloat16), w_ref[...],
                         preferred_element_type=jnp.float32)


def _norm_proj(y2d, z_src, norm_w, w_bf16, *, tm=256):
    M, D = y2d.shape
    _, N = w_bf16.shape
    return pl.pallas_call(
        _norm_proj_kernel,
        out_shape=jax.ShapeDtypeStruct((M, N), jnp.float32),
        grid=(M // tm,),
        in_specs=[
            pl.BlockSpec((tm, D), lambda i: (i, 0)),
            pl.BlockSpec((tm, D), lambda i: (i, 0)),
            pl.BlockSpec((1, D), lambda i: (0, 0)),
            pl.BlockSpec((D, N), lambda i: (0, 0)),
        ],
        out_specs=pl.BlockSpec((tm, N), lambda i: (i, 0)),
        compiler_params=pltpu.CompilerParams(
            dimension_semantics=("parallel",),
            vmem_limit_bytes=56 * 1024 * 1024),
    )(y2d, z_src, norm_w.reshape(1, D), w_bf16)


def kernel(u, in_proj_wT, conv_w_klc, conv_b, A_log, D, dt_bias, norm_w,
           out_proj_wT):
    d_model, d_inner, d_state = 2048, 4096, 128
    H, P, G, K = 64, 64, 1, 4
    HT = 16
    chunk = 128
    n_tiles = H // HT
    conv_dim = d_inner + 2 * G * d_state
    d_in_proj = 2 * d_inner + 2 * G * d_state + H

    Bsz, L, _ = u.shape
    M = Bsz * L

    nx_cols = ((d_in_proj + 511) // 512 - d_inner // 512) * 512
    cwp = jnp.pad(conv_w_klc, ((0, 0), (0, 0), (0, nx_cols - conv_dim)))
    cbp = jnp.pad(conv_b, (0, nx_cols - conv_dim)).reshape(1, -1)
    z2d, xbc2d, dt_pad = _inproj_conv(
        u.reshape(M, d_model).astype(jnp.bfloat16), in_proj_wT, cwp, cbp,
        d_inner=d_inner, seq_len=L)
    xbc = xbc2d.reshape(Bsz, L, -1)

    dt_raw = dt_pad[:, :H]
    dt_t = dt_raw.reshape(Bsz, L, n_tiles, HT).transpose(0, 2, 1, 3)

    A = (-jnp.exp(A_log)).reshape(n_tiles, 1, HT).astype(jnp.float32)
    Dv = D.reshape(n_tiles, 1, HT).astype(jnp.float32)
    dtb = dt_bias.reshape(n_tiles, 1, HT).astype(jnp.float32)

    y = _ssd_scan(xbc, dt_t, A, Dv, dtb, nheads=H, headdim=P,
                  d_state=d_state, chunk=chunk)

    out = _norm_proj(y.reshape(M, d_inner), z2d, norm_w, out_proj_wT)
    return out.reshape(Bsz, L, d_model)


# speedup vs baseline: 1.1590x; 1.0485x over previous
import functools

import jax
import jax.numpy as jnp
from jax import lax
from jax.experimental import pallas as pl
from jax.experimental.pallas import tpu as pltpu


def _sigmoid(x):
    return 1.0 / (1.0 + jnp.exp(-x))


def _sel_dot(x, E):
    xi = lax.bitcast_convert_type(x, jnp.uint32)
    hi = lax.bitcast_convert_type(xi & jnp.uint32(0xFFFF0000), jnp.float32)
    lo = x - hi
    return (jnp.dot(hi, E, preferred_element_type=jnp.float32)
            + jnp.dot(lo, E, preferred_element_type=jnp.float32))


def _softplus(x):
    return jnp.maximum(x, 0.0) + jnp.log(1.0 + jnp.exp(-jnp.abs(x)))


def _inproj_conv_kernel(x_ref, w_ref, cw_ref, cb_ref,
                        oz_ref, oxbc_ref, odt_ref, *,
                        nz_tiles, n_tiles, seq_len, dt_off):
    j = pl.program_id(1)
    acc = jnp.dot(x_ref[...], w_ref[...],
                  preferred_element_type=jnp.float32)

    @pl.when(j < nz_tiles)
    def _store_z():
        oz_ref[...] = acc

    @pl.when(j >= nz_tiles)
    def _conv_silu():
        R, C = acc.shape
        K = cw_ref.shape[0]
        cw = cw_ref[:, 0, :]
        rowmod = lax.broadcasted_iota(jnp.int32, (R, C), 0) & (seq_len - 1)
        total = acc * cw[K - 1:K, :] + cb_ref[...]
        for d in range(1, K):
            sh = jnp.concatenate(
                [jnp.zeros((d, C), jnp.float32), acc[: R - d, :]], axis=0)
            sh = jnp.where(rowmod >= d, sh, 0.0)
            total = total + sh * cw[K - 1 - d:K - d, :]
        oxbc_ref[...] = (total * _sigmoid(total)).astype(oxbc_ref.dtype)

    @pl.when(j == n_tiles - 1)
    def _store_dt():
        odt_ref[...] = acc[:, dt_off:dt_off + odt_ref.shape[1]]


def _inproj_conv(x_bf16, w_bf16, cwp, cbp, *, d_inner, seq_len,
                 tm=2048, tn=512):
    M, K = x_bf16.shape
    _, N = w_bf16.shape
    assert tm % seq_len == 0 and d_inner % tn == 0
    grid_m = M // tm
    grid_n = (N + tn - 1) // tn
    nz = d_inner // tn
    nx = grid_n - nz
    dt_off = (N - 64) - (grid_n - 1) * tn
    kfn = functools.partial(_inproj_conv_kernel, nz_tiles=nz,
                            n_tiles=grid_n, seq_len=seq_len, dt_off=dt_off)
    return pl.pallas_call(
        kfn,
        out_shape=[
            jax.ShapeDtypeStruct((M, d_inner), jnp.float32),
            jax.ShapeDtypeStruct((M, nx * tn), jnp.bfloat16),
            jax.ShapeDtypeStruct((M, 128), jnp.float32),
        ],
        grid=(grid_m, grid_n),
        in_specs=[
            pl.BlockSpec((tm, K), lambda i, j: (i, 0)),
            pl.BlockSpec((K, tn), lambda i, j: (0, j)),
            pl.BlockSpec((4, 1, tn),
                         lambda i, j, nz=nz: (0, 0, jnp.maximum(j - nz, 0))),
            pl.BlockSpec((1, tn),
                         lambda i, j, nz=nz: (0, jnp.maximum(j - nz, 0))),
        ],
        out_specs=[
            pl.BlockSpec((tm, tn),
                         lambda i, j, nz=nz: (i, jnp.minimum(j, nz - 1))),
            pl.BlockSpec((tm, tn),
                         lambda i, j, nz=nz: (i, jnp.maximum(j - nz, 0))),
            pl.BlockSpec((tm, 128), lambda i, j: (i, 0)),
        ],
        compiler_params=pltpu.CompilerParams(
            dimension_semantics=("parallel", "arbitrary"),
            vmem_limit_bytes=56 * 1024 * 1024),
    )(x_bf16, w_bf16, cwp, cbp)


def _ssd_kernel(A_ref, dvp_ref, dtb_ref, dt_ref, xbc_ref, Bm_ref, Cm_ref,
                e1_ref, e2_ref, y_ref, state_ref, xw_ref, *, headdim):
    P = headdim
    HT = A_ref.shape[-1]
    Q = xbc_ref.shape[1]

    @pl.when(pl.program_id(2) == 0)
    def _init():
        state_ref[...] = jnp.zeros_like(state_ref)

    A = A_ref[0]
    DvP = dvp_ref[0]
    dtb = dtb_ref[0]
    dt_raw = dt_ref[0, 0]
    x = xbc_ref[0]
    Bg = Bm_ref[0]
    Cg = Cm_ref[0]
    E1 = e1_ref[...]
    E2 = e2_ref[...]

    dt = _softplus(dt_raw + dtb)
    a = dt * A

    idx_i = lax.broadcasted_iota(jnp.int32, (Q, Q), 0)
    idx_j = lax.broadcasted_iota(jnp.int32, (Q, Q), 1)
    causal = idx_i >= idx_j
    tri = causal.astype(jnp.float32)

    cA = jnp.dot(tri, a, preferred_element_type=jnp.float32)
    cAT = cA.T
    exp_cA = jnp.exp(cA)
    cA_last = cA[Q - 1:Q, :]
    w_all = jnp.exp(cA_last - cA) * dt

    dtP = _sel_dot(dt, E1)
    expP = _sel_dot(exp_cA, E1)
    wP = _sel_dot(w_all, E1)
    elP = expP[Q - 1:Q]
    M2 = _sel_dot(cA, E2)

    xf = x.astype(jnp.float32)
    xdt_bf = (xf * dtP).astype(jnp.bfloat16)
    xw_ref[...] = (xf * wP).astype(jnp.bfloat16)

    BgT = Bg.T
    scores = jnp.dot(Cg, BgT, preferred_element_type=jnp.float32)
    y_inter = jnp.dot(Cg, state_ref[...].astype(jnp.bfloat16),
                      preferred_element_type=jnp.float32)

    neg_big = jnp.float32(-1e30)
    for h in range(HT):
        sl = slice(h * P, (h + 1) * P)
        sq = slice(h * Q, (h + 1) * Q)
        diff = M2[:, sq] - cAT[h:h + 1, :]
        dec = jnp.exp(jnp.where(causal, diff, neg_big))
        y_ref[0, :, sl] = jnp.dot((scores * dec).astype(jnp.bfloat16),
                                  xdt_bf[:, sl],
                                  preferred_element_type=jnp.float32)

    y_ref[0] = y_ref[0] + expP * y_inter + DvP * xf

    dS = jnp.dot(BgT, xw_ref[...], preferred_element_type=jnp.float32)
    state_ref[...] = elP * state_ref[...] + dS


def _ssd_scan(xbc, dt_t, A, Dv, dtb, *, nheads, headdim, d_state, chunk):
    Bsz, L, _ = xbc.shape
    H, P, N, Q = nheads, headdim, d_state, chunk
    d_inner = H * P
    HT = dt_t.shape[-1]
    n_tiles = H // HT
    nC = L // Q
    bcol = d_inner // (HT * P)
    assert d_inner % (HT * P) == 0 and L % Q == 0

    hh = jnp.arange(HT, dtype=jnp.int32)[:, None]
    E1 = (jnp.arange(HT * P, dtype=jnp.int32)[None, :] // P
          == hh).astype(jnp.float32)
    E2 = (jnp.arange(HT * Q, dtype=jnp.int32)[None, :] // Q
          == hh).astype(jnp.float32)
    DvP = jnp.repeat(Dv.reshape(n_tiles, 1, HT), P, axis=2)

    kfn = functools.partial(_ssd_kernel, headdim=P)
    return pl.pallas_call(
        kfn,
        out_shape=jax.ShapeDtypeStruct((Bsz, L, d_inner), jnp.float32),
        grid=(Bsz, n_tiles, nC),
        in_specs=[
            pl.BlockSpec((1, 1, HT), lambda b, t, c: (t, 0, 0)),
            pl.BlockSpec((1, 1, HT * P), lambda b, t, c: (t, 0, 0)),
            pl.BlockSpec((1, 1, HT), lambda b, t, c: (t, 0, 0)),
            pl.BlockSpec((1, 1, Q, HT), lambda b, t, c: (b, t, c, 0)),
            pl.BlockSpec((1, Q, HT * P), lambda b, t, c: (b, c, t)),
            pl.BlockSpec((1, Q, N),
                         lambda b, t, c: (b, c, bcol * (HT * P) // N)),
            pl.BlockSpec((1, Q, N),
                         lambda b, t, c: (b, c, bcol * (HT * P) // N + 1)),
            pl.BlockSpec((HT, HT * P), lambda b, t, c: (0, 0)),
            pl.BlockSpec((HT, HT * Q), lambda b, t, c: (0, 0)),
        ],
        out_specs=pl.BlockSpec((1, Q, HT * P), lambda b, t, c: (b, c, t)),
        scratch_shapes=[pltpu.VMEM((N, HT * P), jnp.float32),
                        pltpu.VMEM((Q, HT * P), jnp.bfloat16)],
        compiler_params=pltpu.CompilerParams(
            dimension_semantics=("parallel", "arbitrary", "arbitrary"),
            vmem_limit_bytes=24 * 1024 * 1024),
    )(A, DvP, dtb, dt_t, xbc, xbc, xbc, E1, E2)


def _norm_proj_kernel(y_ref, z_ref, nw_ref, w_ref, o_ref):
    y = y_ref[...]
    z = z_ref[...]
    x = y * (z * _sigmoid(z))
    var = jnp.mean(x * x, axis=-1, keepdims=True)
    xn = x * lax.rsqrt(var + 1e-5) * nw_ref[...]
    o_ref[...] = jnp.dot(xn.astype(jnp.bfloat16), w_ref[...],
                         preferred_element_type=jnp.float32)


def _norm_proj(y2d, z_src, norm_w, w_bf16, *, tm=256):
    M, D = y2d.shape
    _, N = w_bf16.shape
    return pl.pallas_call(
        _norm_proj_kernel,
        out_shape=jax.ShapeDtypeStruct((M, N), jnp.float32),
        grid=(M // tm,),
        in_specs=[
            pl.BlockSpec((tm, D), lambda i: (i, 0)),
            pl.BlockSpec((tm, D), lambda i: (i, 0)),
            pl.BlockSpec((1, D), lambda i: (0, 0)),
            pl.BlockSpec((D, N), lambda i: (0, 0)),
        ],
        out_specs=pl.BlockSpec((tm, N), lambda i: (i, 0)),
        compiler_params=pltpu.CompilerParams(
            dimension_semantics=("parallel",),
            vmem_limit_bytes=56 * 1024 * 1024),
    )(y2d, z_src, norm_w.reshape(1, D), w_bf16)


def kernel(u, in_proj_wT, conv_w_klc, conv_b, A_log, D, dt_bias, norm_w,
           out_proj_wT):
    d_model, d_inner, d_state = 2048, 4096, 128
    H, P, G, K = 64, 64, 1, 4
    HT = 32
    chunk = 128
    n_tiles = H // HT
    conv_dim = d_inner + 2 * G * d_state
    d_in_proj = 2 * d_inner + 2 * G * d_state + H

    Bsz, L, _ = u.shape
    M = Bsz * L

    nx_cols = ((d_in_proj + 511) // 512 - d_inner // 512) * 512
    cwp = jnp.pad(conv_w_klc, ((0, 0), (0, 0), (0, nx_cols - conv_dim)))
    cbp = jnp.pad(conv_b, (0, nx_cols - conv_dim)).reshape(1, -1)
    z2d, xbc2d, dt_pad = _inproj_conv(
        u.reshape(M, d_model).astype(jnp.bfloat16), in_proj_wT, cwp, cbp,
        d_inner=d_inner, seq_len=L)
    xbc = xbc2d.reshape(Bsz, L, -1)

    dt_raw = dt_pad[:, :H]
    dt_t = dt_raw.reshape(Bsz, L, n_tiles, HT).transpose(0, 2, 1, 3)

    A = (-jnp.exp(A_log)).reshape(n_tiles, 1, HT).astype(jnp.float32)
    Dv = D.reshape(n_tiles, 1, HT).astype(jnp.float32)
    dtb = dt_bias.reshape(n_tiles, 1, HT).astype(jnp.float32)

    y = _ssd_scan(xbc, dt_t, A, Dv, dtb, nheads=H, headdim=P,
                  d_state=d_state, chunk=chunk)

    out = _norm_proj(y.reshape(M, d_inner), z2d, norm_w, out_proj_wT)
    return out.reshape(Bsz, L, d_model)


# speedup vs baseline: 1.1704x; 1.0098x over previous
import functools

import jax
import jax.numpy as jnp
from jax import lax
from jax.experimental import pallas as pl
from jax.experimental.pallas import tpu as pltpu


def _sigmoid(x):
    return 1.0 / (1.0 + jnp.exp(-x))


def _sel_dot(x, E):
    xi = lax.bitcast_convert_type(x, jnp.uint32)
    hi = lax.bitcast_convert_type(xi & jnp.uint32(0xFFFF0000), jnp.float32)
    lo = x - hi
    return (jnp.dot(hi, E, preferred_element_type=jnp.float32)
            + jnp.dot(lo, E, preferred_element_type=jnp.float32))


def _softplus(x):
    return jnp.maximum(x, 0.0) + jnp.log(1.0 + jnp.exp(-jnp.abs(x)))


def _inproj_conv_kernel(x_ref, w_ref, cw_ref, cb_ref,
                        oz_ref, oxbc_ref, odt_ref, *,
                        nz_tiles, n_tiles, seq_len, dt_off):
    j = pl.program_id(1)
    acc = jnp.dot(x_ref[...], w_ref[...],
                  preferred_element_type=jnp.float32)

    @pl.when(j < nz_tiles)
    def _store_z():
        oz_ref[...] = acc

    @pl.when(j >= nz_tiles)
    def _conv_silu():
        R, C = acc.shape
        K = cw_ref.shape[0]
        cw = cw_ref[:, 0, :]
        rowmod = lax.broadcasted_iota(jnp.int32, (R, C), 0) & (seq_len - 1)
        total = acc * cw[K - 1:K, :] + cb_ref[...]
        for d in range(1, K):
            sh = jnp.concatenate(
                [jnp.zeros((d, C), jnp.float32), acc[: R - d, :]], axis=0)
            sh = jnp.where(rowmod >= d, sh, 0.0)
            total = total + sh * cw[K - 1 - d:K - d, :]
        oxbc_ref[...] = (total * _sigmoid(total)).astype(oxbc_ref.dtype)

    @pl.when(j == n_tiles - 1)
    def _store_dt():
        odt_ref[...] = acc[:, dt_off:dt_off + odt_ref.shape[1]]


def _inproj_conv(x_bf16, w_bf16, cwp, cbp, *, d_inner, seq_len,
                 tm=2048, tn=512):
    M, K = x_bf16.shape
    _, N = w_bf16.shape
    assert tm % seq_len == 0 and d_inner % tn == 0
    grid_m = M // tm
    grid_n = (N + tn - 1) // tn
    nz = d_inner // tn
    nx = grid_n - nz
    dt_off = (N - 64) - (grid_n - 1) * tn
    kfn = functools.partial(_inproj_conv_kernel, nz_tiles=nz,
                            n_tiles=grid_n, seq_len=seq_len, dt_off=dt_off)
    return pl.pallas_call(
        kfn,
        out_shape=[
            jax.ShapeDtypeStruct((M, d_inner), jnp.float32),
            jax.ShapeDtypeStruct((M, nx * tn), jnp.bfloat16),
            jax.ShapeDtypeStruct((M, 128), jnp.float32),
        ],
        grid=(grid_m, grid_n),
        in_specs=[
            pl.BlockSpec((tm, K), lambda i, j: (i, 0)),
            pl.BlockSpec((K, tn), lambda i, j: (0, j)),
            pl.BlockSpec((4, 1, tn),
                         lambda i, j, nz=nz: (0, 0, jnp.maximum(j - nz, 0))),
            pl.BlockSpec((1, tn),
                         lambda i, j, nz=nz: (0, jnp.maximum(j - nz, 0))),
        ],
        out_specs=[
            pl.BlockSpec((tm, tn),
                         lambda i, j, nz=nz: (i, jnp.minimum(j, nz - 1))),
            pl.BlockSpec((tm, tn),
                         lambda i, j, nz=nz: (i, jnp.maximum(j - nz, 0))),
            pl.BlockSpec((tm, 128), lambda i, j: (i, 0)),
        ],
        compiler_params=pltpu.CompilerParams(
            dimension_semantics=("parallel", "arbitrary"),
            vmem_limit_bytes=56 * 1024 * 1024),
    )(x_bf16, w_bf16, cwp, cbp)


def _ssd_kernel(A_ref, dvp_ref, dtb_ref, dt_ref, xbc_ref, Bm_ref, Cm_ref,
                e1_ref, e2_ref, y_ref, state_ref, xw_ref, *, headdim):
    P = headdim
    HT = A_ref.shape[-1]
    Q = xbc_ref.shape[1]

    @pl.when(pl.program_id(2) == 0)
    def _init():
        state_ref[...] = jnp.zeros_like(state_ref)

    A = A_ref[0]
    DvP = dvp_ref[0]
    dtb = dtb_ref[0]
    dt_raw = dt_ref[0, 0]
    x = xbc_ref[0]
    Bg = Bm_ref[0]
    Cg = Cm_ref[0]
    E1 = e1_ref[...]
    E2 = e2_ref[...]

    dt = _softplus(dt_raw + dtb)
    a = dt * A

    idx_i = lax.broadcasted_iota(jnp.int32, (Q, Q), 0)
    idx_j = lax.broadcasted_iota(jnp.int32, (Q, Q), 1)
    causal = idx_i >= idx_j
    tri = causal.astype(jnp.float32)

    cA = jnp.dot(tri, a, preferred_element_type=jnp.float32)
    cAT = cA.T
    exp_cA = jnp.exp(cA)
    cA_last = cA[Q - 1:Q, :]
    w_all = jnp.exp(cA_last - cA) * dt

    dtP = _sel_dot(dt, E1)
    expP = _sel_dot(exp_cA, E1)
    wP = _sel_dot(w_all, E1)
    elP = expP[Q - 1:Q]
    M2 = _sel_dot(cA, E2)

    xf = x.astype(jnp.float32)
    xdt_bf = (xf * dtP).astype(jnp.bfloat16)
    xw_ref[...] = (xf * wP).astype(jnp.bfloat16)

    BgT = Bg.T
    scores = jnp.dot(Cg, BgT, preferred_element_type=jnp.float32)
    y_inter = jnp.dot(Cg, state_ref[...].astype(jnp.bfloat16),
                      preferred_element_type=jnp.float32)

    neg_big = jnp.float32(-1e30)
    for h in range(HT):
        sl = slice(h * P, (h + 1) * P)
        sq = slice(h * Q, (h + 1) * Q)
        diff = M2[:, sq] - cAT[h:h + 1, :]
        dec = jnp.exp(jnp.where(causal, diff, neg_big))
        y_ref[0, :, sl] = jnp.dot((scores * dec).astype(jnp.bfloat16),
                                  xdt_bf[:, sl],
                                  preferred_element_type=jnp.float32)

    y_ref[0] = y_ref[0] + expP * y_inter + DvP * xf

    dS = jnp.dot(BgT, xw_ref[...], preferred_element_type=jnp.float32)
    state_ref[...] = elP * state_ref[...] + dS


def _ssd_scan(xbc, dt_t, A, Dv, dtb, *, nheads, headdim, d_state, chunk):
    Bsz, L, _ = xbc.shape
    H, P, N, Q = nheads, headdim, d_state, chunk
    d_inner = H * P
    HT = dt_t.shape[-1]
    n_tiles = H // HT
    nC = L // Q
    bcol = d_inner // (HT * P)
    assert d_inner % (HT * P) == 0 and L % Q == 0

    hh = jnp.arange(HT, dtype=jnp.int32)[:, None]
    E1 = (jnp.arange(HT * P, dtype=jnp.int32)[None, :] // P
          == hh).astype(jnp.float32)
    E2 = (jnp.arange(HT * Q, dtype=jnp.int32)[None, :] // Q
          == hh).astype(jnp.float32)
    DvP = jnp.repeat(Dv.reshape(n_tiles, 1, HT), P, axis=2)

    kfn = functools.partial(_ssd_kernel, headdim=P)
    return pl.pallas_call(
        kfn,
        out_shape=jax.ShapeDtypeStruct((Bsz, L, d_inner), jnp.float32),
        grid=(Bsz, n_tiles, nC),
        in_specs=[
            pl.BlockSpec((1, 1, HT), lambda b, t, c: (t, 0, 0)),
            pl.BlockSpec((1, 1, HT * P), lambda b, t, c: (t, 0, 0)),
            pl.BlockSpec((1, 1, HT), lambda b, t, c: (t, 0, 0)),
            pl.BlockSpec((1, 1, Q, HT), lambda b, t, c: (b, t, c, 0)),
            pl.BlockSpec((1, Q, HT * P), lambda b, t, c: (b, c, t)),
            pl.BlockSpec((1, Q, N),
                         lambda b, t, c: (b, c, bcol * (HT * P) // N)),
            pl.BlockSpec((1, Q, N),
                         lambda b, t, c: (b, c, bcol * (HT * P) // N + 1)),
            pl.BlockSpec((HT, HT * P), lambda b, t, c: (0, 0)),
            pl.BlockSpec((HT, HT * Q), lambda b, t, c: (0, 0)),
        ],
        out_specs=pl.BlockSpec((1, Q, HT * P), lambda b, t, c: (b, c, t)),
        scratch_shapes=[pltpu.VMEM((N, HT * P), jnp.float32),
                        pltpu.VMEM((Q, HT * P), jnp.bfloat16)],
        compiler_params=pltpu.CompilerParams(
            dimension_semantics=("parallel", "arbitrary", "arbitrary"),
            vmem_limit_bytes=24 * 1024 * 1024),
    )(A, DvP, dtb, dt_t, xbc, xbc, xbc, E1, E2)


def _norm_proj_kernel(y_ref, z_ref, nw_ref, w_ref, o_ref):
    y = y_ref[...]
    z = z_ref[...]
    x = y * (z * _sigmoid(z))
    var = jnp.mean(x * x, axis=-1, keepdims=True)
    xn = x * lax.rsqrt(var + 1e-5) * nw_ref[...]
    o_ref[...] = jnp.dot(xn.astype(jnp.bfloat16), w_ref[...],
                         preferred_element_type=jnp.float32)


def _norm_proj(y2d, z_src, norm_w, w_bf16, *, tm=256):
    M, D = y2d.shape
    _, N = w_bf16.shape
    return pl.pallas_call(
        _norm_proj_kernel,
        out_shape=jax.ShapeDtypeStruct((M, N), jnp.float32),
        grid=(M // tm,),
        in_specs=[
            pl.BlockSpec((tm, D), lambda i: (i, 0)),
            pl.BlockSpec((tm, D), lambda i: (i, 0)),
            pl.BlockSpec((1, D), lambda i: (0, 0)),
            pl.BlockSpec((D, N), lambda i: (0, 0)),
        ],
        out_specs=pl.BlockSpec((tm, N), lambda i: (i, 0)),
        compiler_params=pltpu.CompilerParams(
            dimension_semantics=("parallel",),
            vmem_limit_bytes=56 * 1024 * 1024),
    )(y2d, z_src, norm_w.reshape(1, D), w_bf16)


def kernel(u, in_proj_wT, conv_w_klc, conv_b, A_log, D, dt_bias, norm_w,
           out_proj_wT):
    d_model, d_inner, d_state = 2048, 4096, 128
    H, P, G, K = 64, 64, 1, 4
    HT = 64
    chunk = 128
    n_tiles = H // HT
    conv_dim = d_inner + 2 * G * d_state
    d_in_proj = 2 * d_inner + 2 * G * d_state + H

    Bsz, L, _ = u.shape
    M = Bsz * L

    nx_cols = ((d_in_proj + 511) // 512 - d_inner // 512) * 512
    cwp = jnp.pad(conv_w_klc, ((0, 0), (0, 0), (0, nx_cols - conv_dim)))
    cbp = jnp.pad(conv_b, (0, nx_cols - conv_dim)).reshape(1, -1)
    z2d, xbc2d, dt_pad = _inproj_conv(
        u.reshape(M, d_model).astype(jnp.bfloat16), in_proj_wT, cwp, cbp,
        d_inner=d_inner, seq_len=L)
    xbc = xbc2d.reshape(Bsz, L, -1)

    dt_raw = dt_pad[:, :H]
    dt_t = dt_raw.reshape(Bsz, L, n_tiles, HT).transpose(0, 2, 1, 3)

    A = (-jnp.exp(A_log)).reshape(n_tiles, 1, HT).astype(jnp.float32)
    Dv = D.reshape(n_tiles, 1, HT).astype(jnp.float32)
    dtb = dt_bias.reshape(n_tiles, 1, HT).astype(jnp.float32)

    y = _ssd_scan(xbc, dt_t, A, Dv, dtb, nheads=H, headdim=P,
                  d_state=d_state, chunk=chunk)

    out = _norm_proj(y.reshape(M, d_inner), z2d, norm_w, out_proj_wT)
    return out.reshape(Bsz, L, d_model)
